# Initial kernel scaffold; baseline (speedup 1.0000x reference)
import jax
import jax.numpy as jnp
from jax.experimental import pallas as pl


def kernel(indices, center_weight):
    raise NotImplementedError("write your pallas kernel here")


# speedup vs baseline: 6.1057x; 6.1057x over previous
import functools

import jax
import jax.numpy as jnp
from jax import lax
from jax.experimental import pallas as pl
from jax.experimental.pallas import tpu as pltpu
from jax.experimental.pallas import tpu_sc as plsc

VOCAB = 1000
DIM = 128
SEQ = 50
SAMPLES = 4096
NC, NS = 2, 16
NW = NC * NS
SAMP_W = SAMPLES // NW
CH = 4
N_CHUNK = SAMP_W // CH
NBUF = 2
STAGE_PIECES = 8


def _gather_body(idx_hbm, table_hbm, out_hbm, table_sh, idx_v, b0, b1, g0, g1, s0, s1):
    c = lax.axis_index("c")
    s = lax.axis_index("s")
    wid = s * NC + c
    base = wid * SAMP_W

    bufs = (b0, b1)
    gsems = (g0, g1)
    ssems = (s0, s1)

    for g in range(STAGE_PIECES):
        r0 = s * (STAGE_PIECES * 8) + g * 8

        @pl.when(r0 < VOCAB)
        def _():
            pltpu.sync_copy(table_hbm.at[pl.ds(r0, 8)], table_sh.at[pl.ds(r0, 8)])

    pltpu.sync_copy(idx_hbm.at[pl.ds(base, SAMP_W)], idx_v)
    plsc.subcore_barrier()

    def fire_gathers(q, b):
        for t in range(CH):
            pltpu.async_copy(
                table_sh.at[idx_v.at[q * CH + t]], bufs[b].at[t], gsems[b]
            )

    def drain_gathers(q, b):
        for t in range(CH):
            pltpu.make_async_copy(
                table_sh.at[idx_v.at[q * CH + t]], bufs[b].at[t], gsems[b]
            ).wait()

    for b in range(NBUF):
        fire_gathers(b, b)

    @pl.loop(0, N_CHUNK, step=NBUF)
    def _(q):
        for b in range(NBUF):
            drain_gathers(q + b, b)
            pltpu.async_copy(
                bufs[b], out_hbm.at[pl.ds(base + (q + b) * CH, CH)], ssems[b]
            )
        for b in range(NBUF):

            @pl.when(q + NBUF + b < N_CHUNK)
            def _():
                pltpu.make_async_copy(
                    bufs[b], out_hbm.at[pl.ds(base + (q + b) * CH, CH)], ssems[b]
                ).wait()
                fire_gathers(q + NBUF + b, b)

    for b in range(NBUF):
        pltpu.make_async_copy(
            bufs[b],
            out_hbm.at[pl.ds(base + (N_CHUNK - NBUF + b) * CH, CH)],
            ssems[b],
        ).wait()


@jax.jit
def _lookup(idx, table):
    mesh = plsc.VectorSubcoreMesh(
        core_axis_name="c", subcore_axis_name="s", num_cores=NC, num_subcores=NS
    )
    return pl.kernel(
        _gather_body,
        out_type=jax.ShapeDtypeStruct((SAMPLES, SEQ, DIM), jnp.float32),
        mesh=mesh,
        scratch_types=[
            pltpu.VMEM_SHARED((VOCAB, DIM), jnp.float32),
            pltpu.VMEM((SAMP_W, SEQ), jnp.int32),
            pltpu.VMEM((CH, SEQ, DIM), jnp.float32),
            pltpu.VMEM((CH, SEQ, DIM), jnp.float32),
            pltpu.SemaphoreType.DMA,
            pltpu.SemaphoreType.DMA,
            pltpu.SemaphoreType.DMA,
            pltpu.SemaphoreType.DMA,
        ],
    )(idx, table)


def kernel(indices, center_weight):
    return _lookup(indices.astype(jnp.int32), center_weight)


# speedup vs baseline: 6.1075x; 1.0003x over previous
import functools

import jax
import jax.numpy as jnp
from jax import lax
from jax.experimental import pallas as pl
from jax.experimental.pallas import tpu as pltpu
from jax.experimental.pallas import tpu_sc as plsc

VOCAB = 1000
DIM = 128
SEQ = 50
SAMPLES = 4096
NC, NS = 2, 16
NW = NC * NS
SAMP_W = SAMPLES // NW
CH = 4
N_CHUNK = SAMP_W // CH
NBUF = 2
STAGE_PIECES = 8


def _gather_body(idx_hbm, table_hbm, out_hbm, table_sh, idx_v, b0, b1, g0, g1, s0, s1):
    c = lax.axis_index("c")
    s = lax.axis_index("s")
    wid = s * NC + c
    base = wid * SAMP_W

    bufs = (b0, b1)
    gsems = (g0, g1)
    ssems = (s0, s1)

    for g in range(STAGE_PIECES):
        r0 = s * (STAGE_PIECES * 8) + g * 8

        @pl.when(r0 < VOCAB)
        def _():
            pltpu.sync_copy(table_hbm.at[pl.ds(r0, 8)], table_sh.at[pl.ds(r0, 8)])

    pltpu.sync_copy(idx_hbm.at[pl.ds(base, SAMP_W)], idx_v)
    plsc.subcore_barrier()

    def fire_gathers(q, b):
        for t in range(CH):
            pltpu.async_copy(
                table_sh.at[idx_v.at[q * CH + t]], bufs[b].at[t], gsems[b]
            )

    def drain_gathers(q, b):
        for t in range(CH):
            pltpu.make_async_copy(
                table_sh.at[idx_v.at[q * CH + t]], bufs[b].at[t], gsems[b]
            ).wait()

    for b in range(NBUF):
        fire_gathers(b, b)

    @pl.loop(0, N_CHUNK, step=NBUF)
    def _(q):
        for b in range(NBUF):
            drain_gathers(q + b, b)
            pltpu.async_copy(
                bufs[b], out_hbm.at[pl.ds(base + (q + b) * CH, CH)], ssems[b]
            )
        for b in range(NBUF):

            @pl.when(q + NBUF + b < N_CHUNK)
            def _():
                pltpu.make_async_copy(
                    bufs[b], out_hbm.at[pl.ds(base + (q + b) * CH, CH)], ssems[b]
                ).wait()
                fire_gathers(q + NBUF + b, b)

    for b in range(NBUF):
        pltpu.make_async_copy(
            bufs[b],
            out_hbm.at[pl.ds(base + (N_CHUNK - NBUF + b) * CH, CH)],
            ssems[b],
        ).wait()


@jax.jit
def _lookup(idx, table):
    mesh = plsc.VectorSubcoreMesh(
        core_axis_name="c", subcore_axis_name="s", num_cores=NC, num_subcores=NS
    )
    return pl.kernel(
        _gather_body,
        out_type=jax.ShapeDtypeStruct((SAMPLES, SEQ, DIM), jnp.float32),
        mesh=mesh,
        compiler_params=pltpu.CompilerParams(use_tc_tiling_on_sc=True),
        scratch_types=[
            pltpu.VMEM_SHARED((VOCAB, DIM), jnp.float32),
            pltpu.VMEM((SAMP_W, SEQ), jnp.int32),
            pltpu.VMEM((CH, SEQ, DIM), jnp.float32),
            pltpu.VMEM((CH, SEQ, DIM), jnp.float32),
            pltpu.SemaphoreType.DMA,
            pltpu.SemaphoreType.DMA,
            pltpu.SemaphoreType.DMA,
            pltpu.SemaphoreType.DMA,
        ],
    )(idx, table)


def kernel(indices, center_weight):
    return _lookup(indices.astype(jnp.int32), center_weight)


# speedup vs baseline: 14.3916x; 2.3564x over previous
import functools

import jax
import jax.numpy as jnp
from jax import lax
from jax.experimental import pallas as pl
from jax.experimental.pallas import tpu as pltpu
from jax.experimental.pallas import tpu_sc as plsc

VOCAB = 1000
DIM = 128
SEQ = 50
SAMPLES = 4096
NC, NS = 2, 16
NW = NC * NS
SAMP_W = SAMPLES // NW
NBUF = 5
STAGE_PIECES = 8


def _gather_body(idx_hbm, table_hbm, out_hbm, table_sh, idx_v, b0, b1, b2, b3, b4,
                 g0, g1, g2, g3, g4, s0, s1, s2, s3, s4):
    c = lax.axis_index("c")
    s = lax.axis_index("s")
    wid = s * NC + c
    base = wid * SAMP_W

    bufs = (b0, b1, b2, b3, b4)
    gsems = (g0, g1, g2, g3, g4)
    ssems = (s0, s1, s2, s3, s4)

    for g in range(STAGE_PIECES):
        r0 = s * (STAGE_PIECES * 8) + g * 8

        @pl.when(r0 < VOCAB)
        def _():
            pltpu.sync_copy(table_hbm.at[pl.ds(r0, 8)], table_sh.at[pl.ds(r0, 8)])

    pltpu.sync_copy(idx_hbm.at[:, pl.ds(base, SAMP_W)], idx_v)
    plsc.subcore_barrier()

    for b in range(NBUF):
        pltpu.async_copy(table_sh.at[idx_v.at[b]], bufs[b], gsems[b])

    @pl.loop(0, SEQ, step=NBUF)
    def _(t):
        for b in range(NBUF):
            pltpu.make_async_copy(
                table_sh.at[idx_v.at[t + b]], bufs[b], gsems[b]
            ).wait()
            pltpu.async_copy(
                bufs[b], out_hbm.at[t + b, pl.ds(base, SAMP_W)], ssems[b]
            )
        for b in range(NBUF):

            @pl.when(t + NBUF + b < SEQ)
            def _():
                pltpu.make_async_copy(
                    bufs[b], out_hbm.at[t + b, pl.ds(base, SAMP_W)], ssems[b]
                ).wait()
                pltpu.async_copy(
                    table_sh.at[idx_v.at[t + NBUF + b]], bufs[b], gsems[b]
                )

    for b in range(NBUF):
        pltpu.make_async_copy(
            bufs[b],
            out_hbm.at[SEQ - NBUF + b, pl.ds(base, SAMP_W)],
            ssems[b],
        ).wait()


@jax.jit
def _lookup(idx_t, table):
    mesh = plsc.VectorSubcoreMesh(
        core_axis_name="c", subcore_axis_name="s", num_cores=NC, num_subcores=NS
    )
    return pl.kernel(
        _gather_body,
        out_type=jax.ShapeDtypeStruct((SEQ, SAMPLES, DIM), jnp.float32),
        mesh=mesh,
        compiler_params=pltpu.CompilerParams(use_tc_tiling_on_sc=True),
        scratch_types=[
            pltpu.VMEM_SHARED((VOCAB, DIM), jnp.float32),
            pltpu.VMEM((SEQ, SAMP_W), jnp.int32),
        ]
        + [pltpu.VMEM((SAMP_W, DIM), jnp.float32) for _ in range(NBUF)]
        + [pltpu.SemaphoreType.DMA for _ in range(2 * NBUF)],
    )(idx_t, table)


def kernel(indices, center_weight):
    idx_t = indices.astype(jnp.int32).T
    out_t = _lookup(idx_t, center_weight)
    return jnp.transpose(out_t, (1, 0, 2))


# speedup vs baseline: 15.7404x; 1.0937x over previous
import functools

import jax
import jax.numpy as jnp
from jax import lax
from jax.experimental import pallas as pl
from jax.experimental.pallas import tpu as pltpu
from jax.experimental.pallas import tpu_sc as plsc

VOCAB = 1000
DIM = 128
SEQ = 50
SAMPLES = 4096
NC, NS = 2, 16
NW = NC * NS
SAMP_W = SAMPLES // NW
NBUF = 5
STAGE_PIECES = 8


def _gather_body(idx_hbm, table_hbm, out_hbm, table_sh, idx_v, b0, b1, b2, b3, b4,
                 g0, g1, g2, g3, g4, s0, s1, s2, s3, s4):
    c = lax.axis_index("c")
    s = lax.axis_index("s")
    wid = s * NC + c
    base = wid * SAMP_W

    bufs = (b0, b1, b2, b3, b4)
    gsems = (g0, g1, g2, g3, g4)
    ssems = (s0, s1, s2, s3, s4)

    idx_cp = pltpu.async_copy(idx_hbm.at[:, pl.ds(base, SAMP_W)], idx_v, s0)
    for g in range(STAGE_PIECES):
        r0 = s * (STAGE_PIECES * 8) + g * 8

        @pl.when(r0 < VOCAB)
        def _():
            pltpu.async_copy(
                table_hbm.at[pl.ds(r0, 8)], table_sh.at[pl.ds(r0, 8)], g0
            )

    for g in range(STAGE_PIECES):
        r0 = s * (STAGE_PIECES * 8) + g * 8

        @pl.when(r0 < VOCAB)
        def _():
            pltpu.make_async_copy(
                table_hbm.at[pl.ds(r0, 8)], table_sh.at[pl.ds(r0, 8)], g0
            ).wait()

    idx_cp.wait()
    plsc.subcore_barrier()

    for b in range(NBUF):
        pltpu.async_copy(table_sh.at[idx_v.at[b]], bufs[b], gsems[b])

    @pl.loop(0, SEQ, step=NBUF)
    def _(t):
        for b in range(NBUF):
            pltpu.make_async_copy(
                table_sh.at[idx_v.at[t + b]], bufs[b], gsems[b]
            ).wait()
            pltpu.async_copy(
                bufs[b], out_hbm.at[t + b, pl.ds(base, SAMP_W)], ssems[b]
            )
        for b in range(NBUF):

            @pl.when(t + NBUF + b < SEQ)
            def _():
                pltpu.make_async_copy(
                    bufs[b], out_hbm.at[t + b, pl.ds(base, SAMP_W)], ssems[b]
                ).wait()
                pltpu.async_copy(
                    table_sh.at[idx_v.at[t + NBUF + b]], bufs[b], gsems[b]
                )

    for b in range(NBUF):
        pltpu.make_async_copy(
            bufs[b],
            out_hbm.at[SEQ - NBUF + b, pl.ds(base, SAMP_W)],
            ssems[b],
        ).wait()


@jax.jit
def _lookup(idx_t, table):
    mesh = plsc.VectorSubcoreMesh(
        core_axis_name="c", subcore_axis_name="s", num_cores=NC, num_subcores=NS
    )
    return pl.kernel(
        _gather_body,
        out_type=jax.ShapeDtypeStruct((SEQ, SAMPLES, DIM), jnp.float32),
        mesh=mesh,
        compiler_params=pltpu.CompilerParams(use_tc_tiling_on_sc=True),
        scratch_types=[
            pltpu.VMEM_SHARED((VOCAB, DIM), jnp.float32),
            pltpu.VMEM((SEQ, SAMP_W), jnp.int32),
        ]
        + [pltpu.VMEM((SAMP_W, DIM), jnp.float32) for _ in range(NBUF)]
        + [pltpu.SemaphoreType.DMA for _ in range(2 * NBUF)],
    )(idx_t, table)


def kernel(indices, center_weight):
    idx_t = indices.astype(jnp.int32).T
    out_t = _lookup(idx_t, center_weight)
    return jnp.transpose(out_t, (1, 0, 2))


# speedup vs baseline: 15.8013x; 1.0039x over previous
import functools

import jax
import jax.numpy as jnp
from jax import lax
from jax.experimental import pallas as pl
from jax.experimental.pallas import tpu as pltpu
from jax.experimental.pallas import tpu_sc as plsc

VOCAB = 1000
DIM = 128
SEQ = 50
SAMPLES = 4096
NC, NS = 2, 16
NW = NC * NS
SAMP_W = SAMPLES // NW
HALF = 64
N_ROUND = SEQ * 2
NBUF = 10
STAGE_PIECES = 8


def _gather_body(idx_hbm, table_hbm, out_hbm, table_sh, idx_v, *rest):
    c = lax.axis_index("c")
    s = lax.axis_index("s")
    wid = s * NC + c
    base = wid * SAMP_W

    bufs = rest[:NBUF]
    gsems = rest[NBUF : 2 * NBUF]
    ssems = rest[2 * NBUF :]

    idx_cp = pltpu.async_copy(idx_hbm.at[:, pl.ds(base, SAMP_W)], idx_v, ssems[0])
    for g in range(STAGE_PIECES):
        r0 = s * (STAGE_PIECES * 8) + g * 8

        @pl.when(r0 < VOCAB)
        def _():
            pltpu.async_copy(
                table_hbm.at[pl.ds(r0, 8)], table_sh.at[pl.ds(r0, 8)], gsems[0]
            )

    for g in range(STAGE_PIECES):
        r0 = s * (STAGE_PIECES * 8) + g * 8

        @pl.when(r0 < VOCAB)
        def _():
            pltpu.make_async_copy(
                table_hbm.at[pl.ds(r0, 8)], table_sh.at[pl.ds(r0, 8)], gsems[0]
            ).wait()

    idx_cp.wait()
    plsc.subcore_barrier()

    def gather_src(r):
        t, h = r // 2, r % 2
        return table_sh.at[idx_v.at[t, pl.ds(h * HALF, HALF)]]

    def out_dst(r):
        t, h = r // 2, r % 2
        return out_hbm.at[t, pl.ds(base + h * HALF, HALF)]

    for b in range(NBUF):
        pltpu.async_copy(gather_src(b), bufs[b], gsems[b])

    @pl.loop(0, N_ROUND, step=NBUF)
    def _(j):
        for b in range(NBUF):
            pltpu.make_async_copy(gather_src(j + b), bufs[b], gsems[b]).wait()
            pltpu.async_copy(bufs[b], out_dst(j + b), ssems[b])
        for b in range(NBUF):

            @pl.when(j + NBUF + b < N_ROUND)
            def _():
                pltpu.make_async_copy(bufs[b], out_dst(j + b), ssems[b]).wait()
                pltpu.async_copy(gather_src(j + NBUF + b), bufs[b], gsems[b])

    for b in range(NBUF):
        pltpu.make_async_copy(
            bufs[b], out_dst(N_ROUND - NBUF + b), ssems[b]
        ).wait()


@jax.jit
def _lookup(idx_t, table):
    mesh = plsc.VectorSubcoreMesh(
        core_axis_name="c", subcore_axis_name="s", num_cores=NC, num_subcores=NS
    )
    return pl.kernel(
        _gather_body,
        out_type=jax.ShapeDtypeStruct((SEQ, SAMPLES, DIM), jnp.float32),
        mesh=mesh,
        compiler_params=pltpu.CompilerParams(use_tc_tiling_on_sc=True),
        scratch_types=[
            pltpu.VMEM_SHARED((VOCAB, DIM), jnp.float32),
            pltpu.VMEM((SEQ, SAMP_W), jnp.int32),
        ]
        + [pltpu.VMEM((HALF, DIM), jnp.float32) for _ in range(NBUF)]
        + [pltpu.SemaphoreType.DMA for _ in range(2 * NBUF)],
    )(idx_t, table)


def kernel(indices, center_weight):
    idx_t = indices.astype(jnp.int32).T
    out_t = _lookup(idx_t, center_weight)
    return jnp.transpose(out_t, (1, 0, 2))
